# Initial kernel scaffold; baseline (speedup 1.0000x reference)
import jax
import jax.numpy as jnp
from jax.experimental import pallas as pl


def kernel(x, edge_index, comm_ids, comm_emb, W0, b0, W1, b1, W2, b2, Wp, bp):
    raise NotImplementedError("write your pallas kernel here")


# speedup vs baseline: 5.8094x; 5.8094x over previous
import functools

import jax
import jax.numpy as jnp
from jax import lax
from jax.experimental import pallas as pl
from jax.experimental.pallas import tpu as pltpu
from jax.experimental.pallas import tpu_sc as plsc

_N = 10000
_E = 320000
_DIN = 128
_CD = 16
_H = 128
_NP = 10240
_EP = 327680
_NC_CORES = 2
_NSUB = 16
_NW = _NC_CORES * _NSUB
_PW = _EP // _NW
_B = 128
_NB = _PW // _B
_RPS = _NP // _NSUB
_CPW = _NP // _NW
_IC = 16
_NOC = _NB // _IC
_F32 = jnp.float32


def _sc_deg_body(dst_hbm, degp_hbm, dstv, hist):
    c = lax.axis_index("c")
    s = lax.axis_index("s")
    wid = c * _NSUB + s
    pltpu.sync_copy(dst_hbm.at[pl.ds(wid * _PW, _PW)], dstv)
    zeros16 = jnp.zeros((16,), _F32)

    @pl.loop(0, _PW // 16)
    def _zero(i):
        hist[pl.ds(i * 16, 16)] = zeros16

    ones16 = jnp.ones((16,), _F32)

    @pl.loop(0, _PW // 16)
    def _count(i):
        plsc.addupdate_scatter(hist, [dstv[pl.ds(i * 16, 16)]], ones16)

    pltpu.sync_copy(hist, degp_hbm.at[pl.ds(wid * _NP, _NP)])


@functools.cache
def _get_sc_deg():
    return pl.kernel(
        _sc_deg_body,
        out_type=jax.ShapeDtypeStruct((_NW * _NP,), _F32),
        mesh=plsc.VectorSubcoreMesh(core_axis_name="c", subcore_axis_name="s",
                                    num_cores=_NC_CORES, num_subcores=_NSUB),
        scratch_types=[
            pltpu.VMEM((_PW,), jnp.int32),
            pltpu.VMEM((_NP,), _F32),
        ],
        compiler_params=pltpu.CompilerParams(needs_layout_passes=False),
    )


def _sc_comm_body(cid_hbm, cemb_hbm, c_hbm, cidv, crows, sem):
    c = lax.axis_index("c")
    s = lax.axis_index("s")
    wid = c * _NSUB + s
    pltpu.sync_copy(cid_hbm.at[pl.ds(wid * 3 * _B, 3 * _B)], cidv)
    for j in range(3):
        pltpu.async_copy(cemb_hbm.at[cidv.at[pl.ds(j * _B, _B)]],
                         crows.at[pl.ds(j * _B, _B)], sem).wait()
    pltpu.sync_copy(crows.at[pl.ds(0, _CPW)],
                    c_hbm.at[pl.ds(wid * _CPW, _CPW)])


@functools.cache
def _get_sc_comm():
    return pl.kernel(
        _sc_comm_body,
        out_type=jax.ShapeDtypeStruct((_NP, _H), _F32),
        mesh=plsc.VectorSubcoreMesh(core_axis_name="c", subcore_axis_name="s",
                                    num_cores=_NC_CORES, num_subcores=_NSUB),
        scratch_types=[
            pltpu.VMEM((3 * _B,), jnp.int32),
            pltpu.VMEM((3 * _B, _H), _F32),
            pltpu.SemaphoreType.DMA,
        ],
    )


def _sc_edge_body(g_hbm, src_hbm, dst_hbm, z_hbm, p_hbm,
                  idxs, idxd, buf0, buf1, acc, sem0, sem1):
    c = lax.axis_index("c")
    s = lax.axis_index("s")
    wid = c * _NSUB + s
    pltpu.sync_copy(z_hbm, acc.at[pl.ds(s * _RPS, _RPS)])
    plsc.subcore_barrier()

    for t in range(_NOC):
        pltpu.sync_copy(src_hbm.at[pl.ds(wid * _NB + t * _IC, _IC)], idxs)
        pltpu.sync_copy(dst_hbm.at[pl.ds(wid * _NB + t * _IC, _IC)], idxd)
        pltpu.async_copy(g_hbm.at[idxs.at[0]], buf0, sem0)

        @pl.loop(0, _IC, step=2)
        def _edges(b):
            pltpu.async_copy(g_hbm.at[idxs.at[b + 1]], buf1, sem1)
            pltpu.make_async_copy(g_hbm.at[idxs.at[b]], buf0, sem0).wait()
            pltpu.sync_copy(buf0, acc.at[idxd.at[b]], add=True)

            @pl.when(b + 2 < _IC)
            def _():
                pltpu.async_copy(g_hbm.at[idxs.at[b + 2]], buf0, sem0)
            pltpu.make_async_copy(g_hbm.at[idxs.at[b + 1]], buf1, sem1).wait()
            pltpu.sync_copy(buf1, acc.at[idxd.at[b + 1]], add=True)

    plsc.subcore_barrier()

    for u in range(_RPS // _B):
        rows = pl.ds(s * _RPS + u * _B, _B)
        pltpu.sync_copy(acc.at[rows], buf0)
        pltpu.sync_copy(buf0, p_hbm.at[c].at[rows])


@functools.cache
def _get_sc_edge():
    return pl.kernel(
        _sc_edge_body,
        out_type=jax.ShapeDtypeStruct((_NC_CORES, _NP, _H), _F32),
        mesh=plsc.VectorSubcoreMesh(core_axis_name="c", subcore_axis_name="s",
                                    num_cores=_NC_CORES, num_subcores=_NSUB),
        scratch_types=[
            pltpu.VMEM((_IC, _B), jnp.int32),
            pltpu.VMEM((_IC, _B), jnp.int32),
            pltpu.VMEM((_B, _H), _F32),
            pltpu.VMEM((_B, _H), _F32),
            pltpu.VMEM_SHARED((_NP, _H), _F32),
            pltpu.SemaphoreType.DMA,
            pltpu.SemaphoreType.DMA,
        ],
    )


_R = 512
_G = _NP // _R
_DOT = functools.partial(jnp.dot, preferred_element_type=_F32,
                         precision=lax.Precision.HIGHEST)


def _tc1_body(d_ref, x_ref, c_ref, w0x_ref, w0c_ref, dinv_ref, g0_ref):
    deg = jnp.sum(d_ref[...], axis=0) + 1.0
    dinv = lax.rsqrt(deg)
    hw = (_DOT(x_ref[...], w0x_ref[...])
          + _DOT(c_ref[...][:, :_CD], w0c_ref[...]))
    dinv_ref[...] = dinv
    g0_ref[...] = dinv * hw


_tc1 = pl.pallas_call(
    _tc1_body,
    grid=(_G,),
    in_specs=[
        pl.BlockSpec((_NW, _R, 1), lambda i: (0, i, 0)),
        pl.BlockSpec((_R, _DIN), lambda i: (i, 0)),
        pl.BlockSpec((_R, _H), lambda i: (i, 0)),
        pl.BlockSpec((_DIN, _H), lambda i: (0, 0)),
        pl.BlockSpec((_CD, _H), lambda i: (0, 0)),
    ],
    out_specs=[
        pl.BlockSpec((_R, 1), lambda i: (i, 0)),
        pl.BlockSpec((_R, _H), lambda i: (i, 0)),
    ],
    out_shape=[
        jax.ShapeDtypeStruct((_NP, 1), _F32),
        jax.ShapeDtypeStruct((_NP, _H), _F32),
    ],
)


def _tc_mid_body(p_ref, g_ref, dinv_ref, b_ref, w_ref, h_ref, gn_ref):
    dinv = dinv_ref[...]
    agg = dinv * (p_ref[0] + p_ref[1] + g_ref[...])
    h = jnp.maximum(agg + b_ref[...], 0.0)
    h_ref[...] = h
    gn_ref[...] = dinv * _DOT(h, w_ref[...])


_tc_mid = pl.pallas_call(
    _tc_mid_body,
    grid=(_G,),
    in_specs=[
        pl.BlockSpec((_NC_CORES, _R, _H), lambda i: (0, i, 0)),
        pl.BlockSpec((_R, _H), lambda i: (i, 0)),
        pl.BlockSpec((_R, 1), lambda i: (i, 0)),
        pl.BlockSpec((1, _H), lambda i: (0, 0)),
        pl.BlockSpec((_H, _H), lambda i: (0, 0)),
    ],
    out_specs=[
        pl.BlockSpec((_R, _H), lambda i: (i, 0)),
        pl.BlockSpec((_R, _H), lambda i: (i, 0)),
    ],
    out_shape=[
        jax.ShapeDtypeStruct((_NP, _H), _F32),
        jax.ShapeDtypeStruct((_NP, _H), _F32),
    ],
)


def _tc_fin_body(p_ref, g_ref, dinv_ref, b_ref, h1_ref, h2_ref,
                 wp_ref, bp_ref, o_ref):
    dinv = dinv_ref[...]
    h3 = jnp.maximum(dinv * (p_ref[0] + p_ref[1] + g_ref[...]) + b_ref[...],
                     0.0)
    jk = jnp.maximum(jnp.maximum(h1_ref[...], h2_ref[...]), h3)
    o_ref[...] = _DOT(jk, wp_ref[...]) + bp_ref[...]


_tc_fin = pl.pallas_call(
    _tc_fin_body,
    grid=(_G,),
    in_specs=[
        pl.BlockSpec((_NC_CORES, _R, _H), lambda i: (0, i, 0)),
        pl.BlockSpec((_R, _H), lambda i: (i, 0)),
        pl.BlockSpec((_R, 1), lambda i: (i, 0)),
        pl.BlockSpec((1, _H), lambda i: (0, 0)),
        pl.BlockSpec((_R, _H), lambda i: (i, 0)),
        pl.BlockSpec((_R, _H), lambda i: (i, 0)),
        pl.BlockSpec((_H, _H), lambda i: (0, 0)),
        pl.BlockSpec((1, _H), lambda i: (0, 0)),
    ],
    out_specs=pl.BlockSpec((_R, _H), lambda i: (i, 0)),
    out_shape=jax.ShapeDtypeStruct((_NP, _H), _F32),
)


def kernel(x, edge_index, comm_ids, comm_emb, W0, b0, W1, b1, W2, b2, Wp, bp):
    i32 = jnp.int32
    src = edge_index[0].astype(i32)
    dst = edge_index[1].astype(i32)
    pad = jnp.full((_EP - _E,), _NP - 1, i32)
    src2d = jnp.concatenate([src, pad]).reshape(_NW * _NB, _B)
    dst2d = jnp.concatenate([dst, pad]).reshape(_NW * _NB, _B)

    cid = jnp.concatenate([comm_ids.astype(i32), jnp.zeros((_NP - _N,), i32)])
    cid1d = jnp.zeros((_NW, 3 * _B), i32)
    cid1d = cid1d.at[:, :_CPW].set(cid.reshape(_NW, _CPW)).reshape(_NW * 3 * _B)

    x_pad = jnp.concatenate([x, jnp.zeros((_NP - _N, _DIN), _F32)])
    zrows = jnp.zeros((_RPS, _H), _F32)
    dst1d = jnp.concatenate([dst, pad])

    _sc_deg = _get_sc_deg()
    _sc_comm = _get_sc_comm()
    _sc_edge = _get_sc_edge()

    cemb_pad = jnp.zeros((1008, _H), _F32).at[:comm_emb.shape[0], :_CD].set(
        comm_emb.astype(_F32))
    degp = _sc_deg(dst1d)
    c = _sc_comm(cid1d, cemb_pad)
    dinv, g0 = _tc1(degp.reshape(_NW, _NP, 1), x_pad, c, W0[:_DIN], W0[_DIN:])
    p = _sc_edge(g0, src2d, dst2d, zrows)
    h1, g1 = _tc_mid(p, g0, dinv, b0.reshape(1, _H), W1)
    p = _sc_edge(g1, src2d, dst2d, zrows)
    h2, g2 = _tc_mid(p, g1, dinv, b1.reshape(1, _H), W2)
    p = _sc_edge(g2, src2d, dst2d, zrows)
    out = _tc_fin(p, g2, dinv, b2.reshape(1, _H), h1, h2, Wp,
                  bp.reshape(1, _H))
    return out[:_N]


# speedup vs baseline: 18.2348x; 3.1388x over previous
import functools

import jax
import jax.numpy as jnp
from jax import lax
from jax.experimental import pallas as pl
from jax.experimental.pallas import tpu as pltpu
from jax.experimental.pallas import tpu_sc as plsc

_N = 10000
_E = 320000
_DIN = 128
_CD = 16
_H = 128
_NP = 10240
_EP = 327680
_NC_CORES = 2
_NSUB = 16
_NW = _NC_CORES * _NSUB
_PW = _EP // _NW
_B = 128
_NB = _PW // _B
_RPS = _NP // _NSUB
_CPW = _NP // _NW
_IC = 16
_NOC = _NB // _IC
_F32 = jnp.float32


def _sc_deg_body(dst_hbm, degp_hbm, dstv, hist):
    c = lax.axis_index("c")
    s = lax.axis_index("s")
    wid = c * _NSUB + s
    pltpu.sync_copy(dst_hbm.at[pl.ds(wid * _PW, _PW)], dstv)
    zeros16 = jnp.zeros((16,), _F32)

    @pl.loop(0, _PW // 16)
    def _zero(i):
        hist[pl.ds(i * 16, 16)] = zeros16

    ones16 = jnp.ones((16,), _F32)

    @pl.loop(0, _PW // 16)
    def _count(i):
        plsc.addupdate_scatter(hist, [dstv[pl.ds(i * 16, 16)]], ones16)

    pltpu.sync_copy(hist, degp_hbm.at[pl.ds(wid * _NP, _NP)])


@functools.cache
def _get_sc_deg():
    return pl.kernel(
        _sc_deg_body,
        out_type=jax.ShapeDtypeStruct((_NW * _NP,), _F32),
        mesh=plsc.VectorSubcoreMesh(core_axis_name="c", subcore_axis_name="s",
                                    num_cores=_NC_CORES, num_subcores=_NSUB),
        scratch_types=[
            pltpu.VMEM((_PW,), jnp.int32),
            pltpu.VMEM((_NP,), _F32),
        ],
        compiler_params=pltpu.CompilerParams(needs_layout_passes=False),
    )


def _sc_comm_body(cid_hbm, cemb_hbm, c_hbm, cidv, crows, sem):
    c = lax.axis_index("c")
    s = lax.axis_index("s")
    wid = c * _NSUB + s
    pltpu.sync_copy(cid_hbm.at[pl.ds(wid * 3 * _B, 3 * _B)], cidv)
    for j in range(3):
        pltpu.async_copy(cemb_hbm.at[cidv.at[pl.ds(j * _B, _B)]],
                         crows.at[pl.ds(j * _B, _B)], sem).wait()
    pltpu.sync_copy(crows.at[pl.ds(0, _CPW)],
                    c_hbm.at[pl.ds(wid * _CPW, _CPW)])


@functools.cache
def _get_sc_comm():
    return pl.kernel(
        _sc_comm_body,
        out_type=jax.ShapeDtypeStruct((_NP, _H), _F32),
        mesh=plsc.VectorSubcoreMesh(core_axis_name="c", subcore_axis_name="s",
                                    num_cores=_NC_CORES, num_subcores=_NSUB),
        scratch_types=[
            pltpu.VMEM((3 * _B,), jnp.int32),
            pltpu.VMEM((3 * _B, _H), _F32),
            pltpu.SemaphoreType.DMA,
        ],
    )


_EB = 32
_EPIPE = 8
_ENB = _PW // _EB
_EIC = 32
_EOC = _ENB // _EIC


def _sc_edge_body(g_hbm, src_hbm, dst_hbm, z_hbm, p_hbm,
                  idxs, idxd, b0, b1, b2, b3, b4, b5, b6, b7, acc,
                  gs0, gs1, gs2, gs3, gs4, gs5, gs6, gs7,
                  ss0, ss1, ss2, ss3, ss4, ss5, ss6, ss7):
    c = lax.axis_index("c")
    s = lax.axis_index("s")
    wid = c * _NSUB + s
    bufs = (b0, b1, b2, b3, b4, b5, b6, b7)
    gsems = (gs0, gs1, gs2, gs3, gs4, gs5, gs6, gs7)
    ssems = (ss0, ss1, ss2, ss3, ss4, ss5, ss6, ss7)
    pltpu.sync_copy(z_hbm, acc.at[pl.ds(s * _RPS, _RPS)])
    plsc.subcore_barrier()

    @pl.loop(0, _EOC)
    def _chunk(t):
        pltpu.sync_copy(src_hbm.at[pl.ds(wid * _ENB + t * _EIC, _EIC)], idxs)
        pltpu.sync_copy(dst_hbm.at[pl.ds(wid * _ENB + t * _EIC, _EIC)], idxd)
        for k in range(_EPIPE):
            pltpu.async_copy(g_hbm.at[idxs.at[k]], bufs[k], gsems[k])

        @pl.loop(0, _EIC, step=_EPIPE)
        def _round(b):
            for k in range(_EPIPE):
                pltpu.make_async_copy(g_hbm.at[idxs.at[b + k]],
                                      bufs[k], gsems[k]).wait()
                pltpu.async_copy(bufs[k], acc.at[idxd.at[b + k]],
                                 ssems[k], add=True)
            for k in range(_EPIPE):
                pltpu.make_async_copy(bufs[k], acc.at[idxd.at[b + k]],
                                      ssems[k]).wait()

                @pl.when(b + _EPIPE + k < _EIC)
                def _():
                    pltpu.async_copy(g_hbm.at[idxs.at[b + _EPIPE + k]],
                                     bufs[k], gsems[k])

    plsc.subcore_barrier()

    for u in range(_RPS // _EB):
        rows = pl.ds(s * _RPS + u * _EB, _EB)
        pltpu.sync_copy(acc.at[rows], b0)
        pltpu.sync_copy(b0, p_hbm.at[c].at[rows])


@functools.cache
def _get_sc_edge():
    return pl.kernel(
        _sc_edge_body,
        out_type=jax.ShapeDtypeStruct((_NC_CORES, _NP, _H), _F32),
        mesh=plsc.VectorSubcoreMesh(core_axis_name="c", subcore_axis_name="s",
                                    num_cores=_NC_CORES, num_subcores=_NSUB),
        scratch_types=(
            [pltpu.VMEM((_EIC, _EB), jnp.int32)] * 2
            + [pltpu.VMEM((_EB, _H), _F32)] * _EPIPE
            + [pltpu.VMEM_SHARED((_NP, _H), _F32)]
            + [pltpu.SemaphoreType.DMA] * (2 * _EPIPE)
        ),
    )


_R = 1024
_G = _NP // _R
_DOT = functools.partial(jnp.dot, preferred_element_type=_F32,
                         precision=lax.Precision.HIGHEST)


def _tc1_body(d_ref, x_ref, c_ref, w0x_ref, w0c_ref, dinv_ref, g0_ref):
    deg2 = jnp.sum(d_ref[...], axis=0) + 1.0
    dv2 = lax.rsqrt(deg2)
    rr = _R // 128
    rows = lax.broadcasted_iota(jnp.int32, (_R, rr), 0)
    ks = lax.broadcasted_iota(jnp.int32, (_R, rr), 1)
    xsel = ((rows >> 7) == ks).astype(_F32)
    jj = lax.broadcasted_iota(jnp.int32, (_R, 128), 0)
    ll = lax.broadcasted_iota(jnp.int32, (_R, 128), 1)
    msel = ((jj & 127) == ll).astype(_F32)
    dinv = jnp.sum(_DOT(xsel, dv2) * msel, axis=1, keepdims=True)
    hw = (_DOT(x_ref[...], w0x_ref[...])
          + _DOT(c_ref[...][:, :_CD], w0c_ref[...]))
    dinv_ref[...] = dinv
    g0_ref[...] = dinv * hw


_tc1 = pl.pallas_call(
    _tc1_body,
    grid=(_G,),
    in_specs=[
        pl.BlockSpec((_NW, _R // 128, 128), lambda i: (0, i, 0)),
        pl.BlockSpec((_R, _DIN), lambda i: (i, 0)),
        pl.BlockSpec((_R, _H), lambda i: (i, 0)),
        pl.BlockSpec((_DIN, _H), lambda i: (0, 0)),
        pl.BlockSpec((_CD, _H), lambda i: (0, 0)),
    ],
    out_specs=[
        pl.BlockSpec((_R, 1), lambda i: (i, 0)),
        pl.BlockSpec((_R, _H), lambda i: (i, 0)),
    ],
    out_shape=[
        jax.ShapeDtypeStruct((_NP, 1), _F32),
        jax.ShapeDtypeStruct((_NP, _H), _F32),
    ],
)


def _tc_mid_body(p_ref, g_ref, dinv_ref, b_ref, w_ref, h_ref, gn_ref):
    dinv = dinv_ref[...]
    agg = dinv * (p_ref[0] + p_ref[1] + g_ref[...])
    h = jnp.maximum(agg + b_ref[...], 0.0)
    h_ref[...] = h
    gn_ref[...] = dinv * _DOT(h, w_ref[...])


_tc_mid = pl.pallas_call(
    _tc_mid_body,
    grid=(_G,),
    in_specs=[
        pl.BlockSpec((_NC_CORES, _R, _H), lambda i: (0, i, 0)),
        pl.BlockSpec((_R, _H), lambda i: (i, 0)),
        pl.BlockSpec((_R, 1), lambda i: (i, 0)),
        pl.BlockSpec((1, _H), lambda i: (0, 0)),
        pl.BlockSpec((_H, _H), lambda i: (0, 0)),
    ],
    out_specs=[
        pl.BlockSpec((_R, _H), lambda i: (i, 0)),
        pl.BlockSpec((_R, _H), lambda i: (i, 0)),
    ],
    out_shape=[
        jax.ShapeDtypeStruct((_NP, _H), _F32),
        jax.ShapeDtypeStruct((_NP, _H), _F32),
    ],
)


def _tc_fin_body(p_ref, g_ref, dinv_ref, b_ref, h1_ref, h2_ref,
                 wp_ref, bp_ref, o_ref):
    dinv = dinv_ref[...]
    h3 = jnp.maximum(dinv * (p_ref[0] + p_ref[1] + g_ref[...]) + b_ref[...],
                     0.0)
    jk = jnp.maximum(jnp.maximum(h1_ref[...], h2_ref[...]), h3)
    o_ref[...] = _DOT(jk, wp_ref[...]) + bp_ref[...]


_tc_fin = pl.pallas_call(
    _tc_fin_body,
    grid=(_G,),
    in_specs=[
        pl.BlockSpec((_NC_CORES, _R, _H), lambda i: (0, i, 0)),
        pl.BlockSpec((_R, _H), lambda i: (i, 0)),
        pl.BlockSpec((_R, 1), lambda i: (i, 0)),
        pl.BlockSpec((1, _H), lambda i: (0, 0)),
        pl.BlockSpec((_R, _H), lambda i: (i, 0)),
        pl.BlockSpec((_R, _H), lambda i: (i, 0)),
        pl.BlockSpec((_H, _H), lambda i: (0, 0)),
        pl.BlockSpec((1, _H), lambda i: (0, 0)),
    ],
    out_specs=pl.BlockSpec((_R, _H), lambda i: (i, 0)),
    out_shape=jax.ShapeDtypeStruct((_NP, _H), _F32),
)


def kernel(x, edge_index, comm_ids, comm_emb, W0, b0, W1, b1, W2, b2, Wp, bp):
    i32 = jnp.int32
    src = edge_index[0].astype(i32)
    dst = edge_index[1].astype(i32)
    pad = _N + (jnp.arange(_EP - _E, dtype=i32) % (_NP - _N))
    src2d = jnp.concatenate([src, pad]).reshape(_NW * _ENB, _EB)
    dst2d = jnp.concatenate([dst, pad]).reshape(_NW * _ENB, _EB)

    cid = jnp.concatenate([comm_ids.astype(i32), jnp.zeros((_NP - _N,), i32)])
    cid1d = jnp.zeros((_NW, 3 * _B), i32)
    cid1d = cid1d.at[:, :_CPW].set(cid.reshape(_NW, _CPW)).reshape(_NW * 3 * _B)

    x_pad = jnp.concatenate([x, jnp.zeros((_NP - _N, _DIN), _F32)])
    zrows = jnp.zeros((_RPS, _H), _F32)
    dst1d = jnp.concatenate([dst, pad])

    _sc_deg = _get_sc_deg()
    _sc_comm = _get_sc_comm()
    _sc_edge = _get_sc_edge()

    cemb_pad = jnp.zeros((1008, _H), _F32).at[:comm_emb.shape[0], :_CD].set(
        comm_emb.astype(_F32))
    degp = _sc_deg(dst1d)
    c = _sc_comm(cid1d, cemb_pad)
    dinv, g0 = _tc1(degp.reshape(_NW, _NP // 128, 128), x_pad, c,
                    W0[:_DIN], W0[_DIN:])
    p = _sc_edge(g0, src2d, dst2d, zrows)
    h1, g1 = _tc_mid(p, g0, dinv, b0.reshape(1, _H), W1)
    p = _sc_edge(g1, src2d, dst2d, zrows)
    h2, g2 = _tc_mid(p, g1, dinv, b1.reshape(1, _H), W2)
    p = _sc_edge(g2, src2d, dst2d, zrows)
    out = _tc_fin(p, g2, dinv, b2.reshape(1, _H), h1, h2, Wp,
                  bp.reshape(1, _H))
    return out[:_N]
